# baseline (speedup 1.0000x reference)
import functools

import jax
import jax.numpy as jnp
from jax.experimental import pallas as pl
from jax.experimental.pallas import tpu as pltpu

_LANE = 128
_SUBLANE = 8
_N_CORES = 2


def _round_up(x, m):
    return (x + m - 1) // m * m


def _mlp_kernel(x_hbm, w0_ref, b0_ref, w1_ref, b1_ref, w2_ref, b2_ref,
                o_hbm, x_buf, o_buf, in_sem, out_sem, *, chunk_m, n_chunks):
    core = pl.program_id(0)
    base = core * (n_chunks * chunk_m)

    def dma_in(slot, chunk):
        return pltpu.make_async_copy(
            x_hbm.at[pl.ds(base + chunk * chunk_m, chunk_m), :],
            x_buf.at[slot], in_sem.at[slot])

    def dma_out(slot, chunk):
        return pltpu.make_async_copy(
            o_buf.at[slot],
            o_hbm.at[pl.ds(base + chunk * chunk_m, chunk_m), :],
            out_sem.at[slot])

    dma_in(0, 0).start()
    for c in range(n_chunks):
        cur = c % 2
        if c + 1 < n_chunks:
            dma_in((c + 1) % 2, c + 1).start()
        dma_in(cur, c).wait()
        z1 = jnp.dot(x_buf[cur], w0_ref[...],
                     preferred_element_type=jnp.float32) + b0_ref[...]
        h1 = jnp.maximum(z1, 0.0)
        z2 = jnp.dot(h1, w1_ref[...],
                     preferred_element_type=jnp.float32) + b1_ref[...]
        h2 = jnp.maximum(z2, 0.0)
        z3 = jnp.dot(h2, w2_ref[...],
                     preferred_element_type=jnp.float32) + b2_ref[...]
        if c >= 2:
            dma_out(cur, c - 2).wait()
        o_buf[cur] = z3.astype(o_buf.dtype)
        dma_out(cur, c).start()
    for c in range(max(n_chunks - 2, 0), n_chunks):
        dma_out(c % 2, c).wait()


def kernel(x, w0, b0, w1, b1, w2, b2, *, chunk_m=1024):
    M, K = x.shape
    ws = [w0, w1, w2]
    bs = [b0, b1, b2]
    dims = [K] + [w.shape[1] for w in ws]
    pad_dims = [_round_up(d, _LANE) for d in dims]

    x_p = jnp.pad(x, ((0, 0), (0, pad_dims[0] - dims[0])))
    flat_params = []
    for i, (w, b) in enumerate(zip(ws, bs)):
        kin, kout = w.shape
        w_p = jnp.pad(w, ((0, pad_dims[i] - kin),
                          (0, pad_dims[i + 1] - kout)))
        b_p = jnp.pad(b, (0, pad_dims[i + 1] - kout)).reshape(1, pad_dims[i + 1])
        flat_params.extend((w_p, b_p))

    chunk_m = min(_round_up(M, _SUBLANE), chunk_m)
    m_pad = _round_up(M, _N_CORES * chunk_m)
    if m_pad != M:
        x_p = jnp.pad(x_p, ((0, m_pad - M), (0, 0)))
    n_chunks = m_pad // (_N_CORES * chunk_m)
    n_out = pad_dims[-1]

    in_specs = [pl.BlockSpec(memory_space=pl.ANY)]
    for p in flat_params:
        in_specs.append(pl.BlockSpec(p.shape, lambda i: (0, 0)))

    flops = 2 * M * sum(dims[i] * dims[i + 1] for i in range(3))
    bytes_accessed = (
        x_p.size * x_p.dtype.itemsize
        + sum(p.size * p.dtype.itemsize for p in flat_params)
        + M * dims[-1] * 4
    )

    out_p = pl.pallas_call(
        functools.partial(_mlp_kernel, chunk_m=chunk_m, n_chunks=n_chunks),
        out_shape=jax.ShapeDtypeStruct((m_pad, n_out), x.dtype),
        grid=(_N_CORES,),
        in_specs=in_specs,
        out_specs=pl.BlockSpec(memory_space=pl.ANY),
        scratch_shapes=[
            pltpu.VMEM((2, chunk_m, pad_dims[0]), x_p.dtype),
            pltpu.VMEM((2, chunk_m, n_out), x.dtype),
            pltpu.SemaphoreType.DMA((2,)),
            pltpu.SemaphoreType.DMA((2,)),
        ],
        compiler_params=pltpu.CompilerParams(
            dimension_semantics=("parallel",),
        ),
        cost_estimate=pl.CostEstimate(
            flops=flops, transcendentals=0, bytes_accessed=bytes_accessed),
    )(x_p, *flat_params)

    return out_p[:M, : dims[-1]]


# speedup vs baseline: 1.0002x; 1.0002x over previous
import functools

import jax
import jax.numpy as jnp
from jax.experimental import pallas as pl
from jax.experimental.pallas import tpu as pltpu

_LANE = 128
_SUBLANE = 8
_N_CORES = 2


def _round_up(x, m):
    return (x + m - 1) // m * m


def _mlp_kernel(x_hbm, w0_ref, b0_ref, w1_ref, b1_ref, w2_ref, b2_ref,
                o_hbm, x_buf, o_buf, in_sem, out_sem, *, chunk_m, n_chunks):
    core = pl.program_id(0)
    base = core * (n_chunks * chunk_m)

    def dma_in(slot, chunk):
        return pltpu.make_async_copy(
            x_hbm.at[pl.ds(base + chunk * chunk_m, chunk_m), :],
            x_buf.at[slot], in_sem.at[slot])

    def dma_out(slot, chunk):
        return pltpu.make_async_copy(
            o_buf.at[slot],
            o_hbm.at[pl.ds(base + chunk * chunk_m, chunk_m), :],
            out_sem.at[slot])

    dma_in(0, 0).start()
    for c in range(n_chunks):
        cur = c % 2
        if c + 1 < n_chunks:
            dma_in((c + 1) % 2, c + 1).start()
        dma_in(cur, c).wait()
        z1 = jnp.dot(x_buf[cur], w0_ref[...],
                     preferred_element_type=jnp.float32) + b0_ref[...]
        h1 = jnp.maximum(z1, 0.0)
        z2 = jnp.dot(h1, w1_ref[...],
                     preferred_element_type=jnp.float32) + b1_ref[...]
        h2 = jnp.maximum(z2, 0.0)
        z3 = jnp.dot(h2, w2_ref[...],
                     preferred_element_type=jnp.float32) + b2_ref[...]
        if c >= 2:
            dma_out(cur, c - 2).wait()
        o_buf[cur] = z3.astype(o_buf.dtype)
        dma_out(cur, c).start()
    for c in range(max(n_chunks - 2, 0), n_chunks):
        dma_out(c % 2, c).wait()


def kernel(x, w0, b0, w1, b1, w2, b2, *, chunk_m=1024):
    M, K = x.shape
    ws = [w0, w1, w2]
    bs = [b0, b1, b2]
    dims = [K] + [w.shape[1] for w in ws]
    pad_dims = [_round_up(d, _LANE) for d in dims]

    x_p = jnp.pad(x, ((0, 0), (0, pad_dims[0] - dims[0])))
    flat_params = []
    for i, (w, b) in enumerate(zip(ws, bs)):
        kin, kout = w.shape
        w_p = jnp.pad(w, ((0, pad_dims[i] - kin),
                          (0, pad_dims[i + 1] - kout)))
        b_p = jnp.pad(b, (0, pad_dims[i + 1] - kout)).reshape(1, pad_dims[i + 1])
        flat_params.extend((w_p, b_p))

    chunk_m = min(_round_up(M, _SUBLANE), chunk_m)
    m_pad = _round_up(M, _N_CORES * chunk_m)
    if m_pad != M:
        x_p = jnp.pad(x_p, ((0, m_pad - M), (0, 0)))
    n_chunks = m_pad // (_N_CORES * chunk_m)
    n_out = pad_dims[-1]

    in_specs = [pl.BlockSpec(memory_space=pltpu.MemorySpace.HBM)]
    for p in flat_params:
        in_specs.append(pl.BlockSpec(p.shape, lambda i: (0, 0)))

    flops = 2 * M * sum(dims[i] * dims[i + 1] for i in range(3))
    bytes_accessed = (
        x_p.size * x_p.dtype.itemsize
        + sum(p.size * p.dtype.itemsize for p in flat_params)
        + M * dims[-1] * 4
    )

    out_p = pl.pallas_call(
        functools.partial(_mlp_kernel, chunk_m=chunk_m, n_chunks=n_chunks),
        out_shape=jax.ShapeDtypeStruct((m_pad, n_out), x.dtype),
        grid=(_N_CORES,),
        in_specs=in_specs,
        out_specs=pl.BlockSpec(memory_space=pltpu.MemorySpace.HBM),
        scratch_shapes=[
            pltpu.VMEM((2, chunk_m, pad_dims[0]), x_p.dtype),
            pltpu.VMEM((2, chunk_m, n_out), x.dtype),
            pltpu.SemaphoreType.DMA((2,)),
            pltpu.SemaphoreType.DMA((2,)),
        ],
        compiler_params=pltpu.CompilerParams(
            dimension_semantics=("parallel",),
        ),
        cost_estimate=pl.CostEstimate(
            flops=flops, transcendentals=0, bytes_accessed=bytes_accessed),
    )(x_p, *flat_params)

    return out_p[:M, : dims[-1]]


# speedup vs baseline: 1.0865x; 1.0862x over previous
import functools

import jax
import jax.numpy as jnp
from jax.experimental import pallas as pl
from jax.experimental.pallas import tpu as pltpu

_LANE = 128
_SUBLANE = 8


def _round_up(x, m):
    return (x + m - 1) // m * m


def _mlp_kernel(x_ref, w0_ref, b0_ref, w1_ref, b1_ref, w2_ref, b2_ref, o_ref,
                *, chunk_m):
    block_m = x_ref.shape[0]
    for c in range(block_m // chunk_m):
        rows = pl.ds(c * chunk_m, chunk_m)
        z1 = jnp.dot(x_ref[rows, :], w0_ref[...],
                     preferred_element_type=jnp.float32) + b0_ref[...]
        h1 = jnp.maximum(z1, 0.0)
        z2 = jnp.dot(h1, w1_ref[...],
                     preferred_element_type=jnp.float32) + b1_ref[...]
        h2 = jnp.maximum(z2, 0.0)
        z3 = jnp.dot(h2, w2_ref[...],
                     preferred_element_type=jnp.float32) + b2_ref[...]
        o_ref[rows, :] = z3.astype(o_ref.dtype)


def kernel(x, w0, b0, w1, b1, w2, b2, *, block_m=1024, chunk_m=1024):
    M, K = x.shape
    ws = [w0, w1, w2]
    bs = [b0, b1, b2]
    dims = [K] + [w.shape[1] for w in ws]
    pad_dims = [_round_up(d, _LANE) for d in dims]

    x_p = jnp.pad(x, ((0, 0), (0, pad_dims[0] - dims[0])))
    flat_params = []
    for i, (w, b) in enumerate(zip(ws, bs)):
        kin, kout = w.shape
        w_p = jnp.pad(w, ((0, pad_dims[i] - kin),
                          (0, pad_dims[i + 1] - kout)))
        b_p = jnp.pad(b, (0, pad_dims[i + 1] - kout)).reshape(1, pad_dims[i + 1])
        flat_params.extend((w_p, b_p))

    block_m = min(_round_up(M, _SUBLANE), block_m)
    chunk_m = min(chunk_m, block_m)
    if block_m % chunk_m:
        chunk_m = block_m
    m_pad = _round_up(M, block_m)
    if m_pad != M:
        x_p = jnp.pad(x_p, ((0, m_pad - M), (0, 0)))
    grid_m = m_pad // block_m

    in_specs = [pl.BlockSpec((block_m, pad_dims[0]), lambda i: (i, 0))]
    for p in flat_params:
        in_specs.append(pl.BlockSpec(p.shape, lambda i: (0, 0)))

    flops = 2 * M * sum(dims[i] * dims[i + 1] for i in range(3))
    bytes_accessed = (
        x_p.size * x_p.dtype.itemsize
        + sum(p.size * p.dtype.itemsize for p in flat_params)
        + M * dims[-1] * 4
    )

    out_p = pl.pallas_call(
        functools.partial(_mlp_kernel, chunk_m=chunk_m),
        out_shape=jax.ShapeDtypeStruct((m_pad, pad_dims[-1]), x.dtype),
        grid=(grid_m,),
        in_specs=in_specs,
        out_specs=pl.BlockSpec((block_m, pad_dims[-1]), lambda i: (i, 0)),
        compiler_params=pltpu.CompilerParams(
            dimension_semantics=("parallel",),
        ),
        cost_estimate=pl.CostEstimate(
            flops=flops, transcendentals=0, bytes_accessed=bytes_accessed),
    )(x_p, *flat_params)

    return out_p[:M, : dims[-1]]
